# baseline (speedup 1.0000x reference)
import functools

import jax
import jax.numpy as jnp
from jax import lax
from jax.experimental import pallas as pl
from jax.experimental.pallas import tpu as pltpu
from jax.experimental.pallas import tpu_sc as plsc

N, E, D_IN, D_HID, D_OUT = 10000, 320000, 128, 256, 128

CHUNK = 128
E_PAD = 327680
NSUB = 16
NCORE = 2
EPS = E_PAD // NSUB
NCH = EPS // CHUNK
GB = 8
NGRP = NCH // GB
EPW = E_PAD // (NSUB * NCORE)
NCH_DEG = EPW // CHUNK

ACC_ROWS = 10112
RPS = ACC_ROWS // NSUB
DEG_ROWS = 10240
DPS = DEG_ROWS // NSUB

_mesh = functools.partial(
    plsc.VectorSubcoreMesh, core_axis_name="c", subcore_axis_name="s")


@functools.partial(
    pl.kernel,
    out_type=jax.ShapeDtypeStruct((NCORE, DEG_ROWS), jnp.float32),
    mesh=_mesh(),
    scratch_types=[
        pltpu.VMEM_SHARED((DEG_ROWS,), jnp.float32),
        pltpu.VMEM((NCH_DEG, CHUNK), jnp.int32),
        pltpu.VMEM((CHUNK,), jnp.float32),
    ],
)
def _sc_degree(dst_hbm, zer_hbm, out_hbm, acc, idx_v, ones_v):
    cid = lax.axis_index("c")
    sid = lax.axis_index("s")
    wid = sid * NCORE + cid
    pltpu.sync_copy(zer_hbm, acc.at[pl.ds(sid * DPS, DPS)])
    pltpu.sync_copy(dst_hbm.at[wid], idx_v)
    for j in range(CHUNK // 16):
        ones_v[pl.ds(j * 16, 16)] = jnp.full((16,), 1.0, jnp.float32)
    plsc.subcore_barrier()

    @pl.loop(0, NCH_DEG)
    def _(i):
        pltpu.sync_copy(ones_v, acc.at[idx_v.at[i]], add=True)

    plsc.subcore_barrier()
    pltpu.sync_copy(acc.at[pl.ds(sid * DPS, DPS)],
                    out_hbm.at[cid, pl.ds(sid * DPS, DPS)])


def _make_sc_aggregate(dh):
    @functools.partial(
        pl.kernel,
        out_type=(jax.ShapeDtypeStruct((ACC_ROWS, dh), jnp.float32),
                  jax.ShapeDtypeStruct((ACC_ROWS, dh), jnp.float32)),
        mesh=_mesh(),
        scratch_types=[
            pltpu.VMEM_SHARED((ACC_ROWS, dh), jnp.float32),
            pltpu.VMEM((GB, CHUNK), jnp.int32),
            pltpu.VMEM((GB, CHUNK), jnp.int32),
            pltpu.VMEM((CHUNK, dh), jnp.float32),
            pltpu.VMEM((CHUNK, dh), jnp.float32),
            pltpu.SemaphoreType.DMA,
            pltpu.SemaphoreType.DMA,
        ],
    )
    def agg(ga_hbm, gb_hbm, src_hbm, dst_hbm, zer_hbm, outa_hbm, outb_hbm,
            acc, src_v, dst_v, rows0, rows1, gsem, ssem):
        cid = lax.axis_index("c")
        sid = lax.axis_index("s")
        pltpu.sync_copy(zer_hbm, acc.at[pl.ds(sid * RPS, RPS)])
        plsc.subcore_barrier()

        def half(g_hbm, out_hbm):
            @pl.loop(0, NGRP)
            def _(g):
                pltpu.sync_copy(src_hbm.at[sid, pl.ds(g * GB, GB)], src_v)
                pltpu.sync_copy(dst_hbm.at[sid, pl.ds(g * GB, GB)], dst_v)
                bufs = (rows0, rows1)
                gcp = [None] * GB
                scp = [None] * GB
                gcp[0] = pltpu.async_copy(g_hbm.at[src_v.at[0]], bufs[0], gsem)
                for j in range(GB):
                    b = bufs[j % 2]
                    gcp[j].wait()
                    scp[j] = pltpu.async_copy(
                        b, acc.at[dst_v.at[j]], ssem, add=True)
                    if j + 1 < GB:
                        if j >= 1:
                            scp[j - 1].wait()
                        gcp[j + 1] = pltpu.async_copy(
                            g_hbm.at[src_v.at[j + 1]], bufs[(j + 1) % 2], gsem)
                scp[GB - 2].wait()
                scp[GB - 1].wait()

            plsc.subcore_barrier()
            pltpu.sync_copy(acc.at[pl.ds(sid * RPS, RPS)],
                            out_hbm.at[pl.ds(sid * RPS, RPS)])

        @pl.when(cid == 0)
        def _():
            half(ga_hbm, outa_hbm)

        @pl.when(cid == 1)
        def _():
            half(gb_hbm, outb_hbm)

    return agg


_sc_agg_hid = _make_sc_aggregate(D_HID // 2)

NGRP2 = NCH_DEG // GB


@functools.partial(
    pl.kernel,
    out_type=(jax.ShapeDtypeStruct((ACC_ROWS, D_OUT), jnp.float32),
              jax.ShapeDtypeStruct((ACC_ROWS, D_OUT), jnp.float32)),
    mesh=_mesh(),
    scratch_types=[
        pltpu.VMEM_SHARED((ACC_ROWS, D_OUT), jnp.float32),
        pltpu.VMEM((GB, CHUNK), jnp.int32),
        pltpu.VMEM((GB, CHUNK), jnp.int32),
        pltpu.VMEM((CHUNK, D_OUT), jnp.float32),
        pltpu.VMEM((CHUNK, D_OUT), jnp.float32),
        pltpu.SemaphoreType.DMA,
        pltpu.SemaphoreType.DMA,
    ],
)
def _sc_agg_out(g_hbm, src_hbm, dst_hbm, zer_hbm, out0_hbm, out1_hbm,
                acc, src_v, dst_v, rows0, rows1, gsem, ssem):
    cid = lax.axis_index("c")
    sid = lax.axis_index("s")
    wid = sid * NCORE + (1 - cid)
    pltpu.sync_copy(zer_hbm, acc.at[pl.ds(sid * RPS, RPS)])
    plsc.subcore_barrier()

    @pl.loop(0, NGRP2)
    def _(g):
        pltpu.sync_copy(src_hbm.at[wid, pl.ds(g * GB, GB)], src_v)
        pltpu.sync_copy(dst_hbm.at[wid, pl.ds(g * GB, GB)], dst_v)
        bufs = (rows0, rows1)
        gcp = [None] * GB
        scp = [None] * GB
        gcp[0] = pltpu.async_copy(g_hbm.at[src_v.at[0]], bufs[0], gsem)
        for j in range(GB):
            b = bufs[j % 2]
            gcp[j].wait()
            scp[j] = pltpu.async_copy(b, acc.at[dst_v.at[j]], ssem, add=True)
            if j + 1 < GB:
                if j >= 1:
                    scp[j - 1].wait()
                gcp[j + 1] = pltpu.async_copy(
                    g_hbm.at[src_v.at[j + 1]], bufs[(j + 1) % 2], gsem)
        scp[GB - 2].wait()
        scp[GB - 1].wait()

    plsc.subcore_barrier()

    @pl.when(cid == 0)
    def _():
        pltpu.sync_copy(acc.at[pl.ds(sid * RPS, RPS)],
                        out0_hbm.at[pl.ds(sid * RPS, RPS)])

    @pl.when(cid == 1)
    def _():
        pltpu.sync_copy(acc.at[pl.ds(sid * RPS, RPS)],
                        out1_hbm.at[pl.ds(sid * RPS, RPS)])


_BR = 400
_GRID = N // _BR


def _dis_of(d_ref):
    return lax.rsqrt(d_ref[:, 0:1] + d_ref[:, 1:2] + 1.0)


def _tc1_body(d_ref, x_ref, w_ref, ga_ref, gb_ref):
    dis = _dis_of(d_ref)
    g = jnp.dot(x_ref[:], w_ref[:], preferred_element_type=jnp.float32) * dis
    ga_ref[:] = g[:, :D_HID // 2]
    gb_ref[:] = g[:, D_HID // 2:]


def _tc2_body(d_ref, sa_ref, sb_ref, ga_ref, gb_ref, b_ref, w_ref, o_ref):
    dis = _dis_of(d_ref)
    ha = jnp.maximum((sa_ref[:] + ga_ref[:]) * dis + b_ref[0, :D_HID // 2], 0.0)
    hb = jnp.maximum((sb_ref[:] + gb_ref[:]) * dis + b_ref[0, D_HID // 2:], 0.0)
    h = jnp.concatenate([ha, hb], axis=1)
    o_ref[:] = jnp.dot(h, w_ref[:], preferred_element_type=jnp.float32) * dis


def _tc3_body(d_ref, s0_ref, s1_ref, g_ref, b_ref, z_ref):
    dis = _dis_of(d_ref)
    z_ref[:] = (s0_ref[:] + s1_ref[:] + g_ref[:]) * dis + b_ref[0, :]


def _row_spec(c):
    return pl.BlockSpec((_BR, c), lambda i: (i, 0))


def _full_spec(r, c):
    return pl.BlockSpec((r, c), lambda i: (0, 0))


_tc1 = pl.pallas_call(
    _tc1_body,
    grid=(_GRID,),
    in_specs=[_row_spec(2), _row_spec(D_IN), _full_spec(D_IN, D_HID)],
    out_specs=[_row_spec(D_HID // 2)] * 2,
    out_shape=[jax.ShapeDtypeStruct((N, D_HID // 2), jnp.float32)] * 2,
)

_tc2 = pl.pallas_call(
    _tc2_body,
    grid=(_GRID,),
    in_specs=[_row_spec(2)] + [_row_spec(D_HID // 2)] * 4
    + [_full_spec(1, D_HID), _full_spec(D_HID, D_OUT)],
    out_specs=_row_spec(D_OUT),
    out_shape=jax.ShapeDtypeStruct((N, D_OUT), jnp.float32),
)

_tc3 = pl.pallas_call(
    _tc3_body,
    grid=(_GRID,),
    in_specs=[_row_spec(2)] + [_row_spec(D_OUT)] * 3
    + [_full_spec(1, D_OUT)],
    out_specs=_row_spec(D_OUT),
    out_shape=jax.ShapeDtypeStruct((N, D_OUT), jnp.float32),
)


def kernel(x, edge_index, W1, b1, W2, b2):
    src = edge_index[0].astype(jnp.int32)
    dst = edge_index[1].astype(jnp.int32)
    pad = E_PAD - E
    src_p = jnp.concatenate([src, jnp.zeros((pad,), jnp.int32)])
    trash = N + jnp.arange(pad, dtype=jnp.int32) % (ACC_ROWS - N)
    dst_p = jnp.concatenate([dst, trash])
    src3 = src_p.reshape(NSUB, NCH, CHUNK)
    dst3 = dst_p.reshape(NSUB, NCH, CHUNK)
    srcw = src_p.reshape(NSUB * NCORE, NCH_DEG, CHUNK)
    dstw = dst_p.reshape(NSUB * NCORE, NCH_DEG, CHUNK)

    zer_deg = jnp.zeros((DPS,), jnp.float32)
    zer_hid = jnp.zeros((RPS, D_HID // 2), jnp.float32)
    zer_out = jnp.zeros((RPS, D_OUT), jnp.float32)

    deg2 = _sc_degree(dstw, zer_deg)
    dpair = deg2.T[:N]

    g1a, g1b = _tc1(dpair, x, W1)
    s1a, s1b = _sc_agg_hid(g1a, g1b, src3, dst3, zer_hid)
    g2 = _tc2(dpair, s1a[:N], s1b[:N], g1a, g1b, b1.reshape(1, D_HID), W2)
    s20, s21 = _sc_agg_out(g2, srcw, dstw, zer_out)
    z = _tc3(dpair, s20[:N], s21[:N], g2, b2.reshape(1, D_OUT))
    return z


# speedup vs baseline: 2.3025x; 2.3025x over previous
import functools

import jax
import jax.numpy as jnp
from jax import lax
from jax.experimental import pallas as pl
from jax.experimental.pallas import tpu as pltpu
from jax.experimental.pallas import tpu_sc as plsc

N, E, D_IN, D_HID, D_OUT = 10000, 320000, 128, 256, 128

CHUNK = 100
NSUB = 16
NCORE = 2
NCH = E // NSUB // CHUNK
GB = 8
NGRP = NCH // GB
NCHW = E // (NSUB * NCORE) // CHUNK

ACC_ROWS = 10112
RPS = ACC_ROWS // NSUB
DEG_ROWS = 10240
DPS = DEG_ROWS // NSUB

_mesh = functools.partial(
    plsc.VectorSubcoreMesh, core_axis_name="c", subcore_axis_name="s")


@functools.partial(
    pl.kernel,
    out_type=jax.ShapeDtypeStruct((NCORE, DEG_ROWS), jnp.float32),
    mesh=_mesh(),
    scratch_types=[
        pltpu.VMEM_SHARED((DEG_ROWS,), jnp.float32),
        pltpu.VMEM((NCHW, CHUNK), jnp.int32),
        pltpu.VMEM((CHUNK,), jnp.float32),
    ],
)
def _sc_degree(dst_hbm, zer_hbm, ones_hbm, out_hbm, acc, idx_v, ones_v):
    cid = lax.axis_index("c")
    sid = lax.axis_index("s")
    wid = sid * NCORE + cid
    pltpu.sync_copy(zer_hbm, acc.at[pl.ds(sid * DPS, DPS)])
    pltpu.sync_copy(dst_hbm.at[wid], idx_v)
    pltpu.sync_copy(ones_hbm, ones_v)
    plsc.subcore_barrier()

    @pl.loop(0, NCHW)
    def _(i):
        pltpu.sync_copy(ones_v, acc.at[idx_v.at[i]], add=True)

    plsc.subcore_barrier()
    pltpu.sync_copy(acc.at[pl.ds(sid * DPS, DPS)],
                    out_hbm.at[cid, pl.ds(sid * DPS, DPS)])


def _make_sc_aggregate(dh):
    @functools.partial(
        pl.kernel,
        out_type=(jax.ShapeDtypeStruct((ACC_ROWS, dh), jnp.float32),
                  jax.ShapeDtypeStruct((ACC_ROWS, dh), jnp.float32)),
        mesh=_mesh(),
        scratch_types=[
            pltpu.VMEM_SHARED((ACC_ROWS, dh), jnp.float32),
            pltpu.VMEM((GB, CHUNK), jnp.int32),
            pltpu.VMEM((GB, CHUNK), jnp.int32),
            pltpu.VMEM((CHUNK, dh), jnp.float32),
            pltpu.VMEM((CHUNK, dh), jnp.float32),
            pltpu.SemaphoreType.DMA,
            pltpu.SemaphoreType.DMA,
        ],
    )
    def agg(ga_hbm, gb_hbm, src_hbm, dst_hbm, zer_hbm, outa_hbm, outb_hbm,
            acc, src_v, dst_v, rows0, rows1, gsem, ssem):
        cid = lax.axis_index("c")
        sid = lax.axis_index("s")
        pltpu.sync_copy(zer_hbm, acc.at[pl.ds(sid * RPS, RPS)])
        plsc.subcore_barrier()

        def half(g_hbm, out_hbm):
            @pl.loop(0, NGRP)
            def _(g):
                pltpu.sync_copy(src_hbm.at[sid, g], src_v)
                pltpu.sync_copy(dst_hbm.at[sid, g], dst_v)
                bufs = (rows0, rows1)
                gcp = [None] * GB
                scp = [None] * GB
                gcp[0] = pltpu.async_copy(g_hbm.at[src_v.at[0]], bufs[0], gsem)
                for j in range(GB):
                    b = bufs[j % 2]
                    gcp[j].wait()
                    scp[j] = pltpu.async_copy(
                        b, acc.at[dst_v.at[j]], ssem, add=True)
                    if j + 1 < GB:
                        if j >= 1:
                            scp[j - 1].wait()
                        gcp[j + 1] = pltpu.async_copy(
                            g_hbm.at[src_v.at[j + 1]], bufs[(j + 1) % 2], gsem)
                scp[GB - 2].wait()
                scp[GB - 1].wait()

            plsc.subcore_barrier()
            pltpu.sync_copy(acc.at[pl.ds(sid * RPS, RPS)],
                            out_hbm.at[pl.ds(sid * RPS, RPS)])

        @pl.when(cid == 0)
        def _():
            half(ga_hbm, outa_hbm)

        @pl.when(cid == 1)
        def _():
            half(gb_hbm, outb_hbm)

    return agg


_sc_agg_hid = _make_sc_aggregate(D_HID // 2)

GB2 = 4
NGRP2 = NCHW // GB2


@functools.partial(
    pl.kernel,
    out_type=(jax.ShapeDtypeStruct((ACC_ROWS, D_OUT), jnp.float32),
              jax.ShapeDtypeStruct((ACC_ROWS, D_OUT), jnp.float32)),
    mesh=_mesh(),
    scratch_types=[
        pltpu.VMEM_SHARED((ACC_ROWS, D_OUT), jnp.float32),
        pltpu.VMEM((GB2, CHUNK), jnp.int32),
        pltpu.VMEM((GB2, CHUNK), jnp.int32),
        pltpu.VMEM((CHUNK, D_OUT), jnp.float32),
        pltpu.VMEM((CHUNK, D_OUT), jnp.float32),
        pltpu.SemaphoreType.DMA,
        pltpu.SemaphoreType.DMA,
    ],
)
def _sc_agg_out(g_hbm, src_hbm, dst_hbm, zer_hbm, out0_hbm, out1_hbm,
                acc, src_v, dst_v, rows0, rows1, gsem, ssem):
    cid = lax.axis_index("c")
    sid = lax.axis_index("s")
    wid = sid * NCORE + cid
    pltpu.sync_copy(zer_hbm, acc.at[pl.ds(sid * RPS, RPS)])
    plsc.subcore_barrier()

    @pl.loop(0, NGRP2)
    def _(g):
        pltpu.sync_copy(src_hbm.at[wid, g], src_v)
        pltpu.sync_copy(dst_hbm.at[wid, g], dst_v)
        bufs = (rows0, rows1)
        gcp = [None] * GB2
        scp = [None] * GB2
        gcp[0] = pltpu.async_copy(g_hbm.at[src_v.at[0]], bufs[0], gsem)
        for j in range(GB2):
            b = bufs[j % 2]
            gcp[j].wait()
            scp[j] = pltpu.async_copy(b, acc.at[dst_v.at[j]], ssem,
                                      add=True)
            if j + 1 < GB2:
                if j >= 1:
                    scp[j - 1].wait()
                gcp[j + 1] = pltpu.async_copy(
                    g_hbm.at[src_v.at[j + 1]], bufs[(j + 1) % 2], gsem)
        scp[GB2 - 2].wait()
        scp[GB2 - 1].wait()

    plsc.subcore_barrier()

    @pl.when(cid == 0)
    def _():
        pltpu.sync_copy(acc.at[pl.ds(sid * RPS, RPS)],
                        out0_hbm.at[pl.ds(sid * RPS, RPS)])

    @pl.when(cid == 1)
    def _():
        pltpu.sync_copy(acc.at[pl.ds(sid * RPS, RPS)],
                        out1_hbm.at[pl.ds(sid * RPS, RPS)])


_BR = 400
_GRID = N // _BR


def _dis_of(d_ref):
    return lax.rsqrt(d_ref[:, 0:1] + d_ref[:, 1:2] + 1.0)


def _tc1_body(d_ref, x_ref, w_ref, ga_ref, gb_ref):
    dis = _dis_of(d_ref)
    g = jnp.dot(x_ref[:], w_ref[:], preferred_element_type=jnp.float32) * dis
    ga_ref[:] = g[:, :D_HID // 2]
    gb_ref[:] = g[:, D_HID // 2:]


def _tc2_body(d_ref, sa_ref, sb_ref, ga_ref, gb_ref, b_ref, w_ref, o_ref):
    dis = _dis_of(d_ref)
    ha = jnp.maximum((sa_ref[:] + ga_ref[:]) * dis + b_ref[0, :D_HID // 2], 0.0)
    hb = jnp.maximum((sb_ref[:] + gb_ref[:]) * dis + b_ref[0, D_HID // 2:], 0.0)
    h = jnp.concatenate([ha, hb], axis=1)
    o_ref[:] = jnp.dot(h, w_ref[:], preferred_element_type=jnp.float32) * dis


def _tc3_body(d_ref, s0_ref, s1_ref, g_ref, b_ref, z_ref):
    dis = _dis_of(d_ref)
    z_ref[:] = (s0_ref[:] + s1_ref[:] + g_ref[:]) * dis + b_ref[0, :]


def _row_spec(c):
    return pl.BlockSpec((_BR, c), lambda i: (i, 0))


def _full_spec(r, c):
    return pl.BlockSpec((r, c), lambda i: (0, 0))


_tc1 = pl.pallas_call(
    _tc1_body,
    grid=(_GRID,),
    in_specs=[_row_spec(2), _row_spec(D_IN), _full_spec(D_IN, D_HID)],
    out_specs=[_row_spec(D_HID // 2)] * 2,
    out_shape=[jax.ShapeDtypeStruct((N, D_HID // 2), jnp.float32)] * 2,
)

_tc2 = pl.pallas_call(
    _tc2_body,
    grid=(_GRID,),
    in_specs=[_row_spec(2)] + [_row_spec(D_HID // 2)] * 4
    + [_full_spec(1, D_HID), _full_spec(D_HID, D_OUT)],
    out_specs=_row_spec(D_OUT),
    out_shape=jax.ShapeDtypeStruct((N, D_OUT), jnp.float32),
)

_tc3 = pl.pallas_call(
    _tc3_body,
    grid=(_GRID,),
    in_specs=[_row_spec(2)] + [_row_spec(D_OUT)] * 3
    + [_full_spec(1, D_OUT)],
    out_specs=_row_spec(D_OUT),
    out_shape=jax.ShapeDtypeStruct((N, D_OUT), jnp.float32),
)


def kernel(x, edge_index, W1, b1, W2, b2):
    src = edge_index[0].astype(jnp.int32)
    dst = edge_index[1].astype(jnp.int32)
    src3 = src.reshape(NSUB, NGRP, GB, CHUNK)
    dst3 = dst.reshape(NSUB, NGRP, GB, CHUNK)
    srcw = src.reshape(NSUB * NCORE, NGRP2, GB2, CHUNK)
    dstw = dst.reshape(NSUB * NCORE, NGRP2, GB2, CHUNK)
    dstd = dst.reshape(NSUB * NCORE, NCHW, CHUNK)

    zer_deg = jnp.zeros((DPS,), jnp.float32)
    ones_c = jnp.ones((CHUNK,), jnp.float32)
    zer_hid = jnp.zeros((RPS, D_HID // 2), jnp.float32)
    zer_out = jnp.zeros((RPS, D_OUT), jnp.float32)

    deg2 = _sc_degree(dstd, zer_deg, ones_c)
    dpair = deg2.T[:N]

    g1a, g1b = _tc1(dpair, x, W1)
    s1a, s1b = _sc_agg_hid(g1a, g1b, src3, dst3, zer_hid)
    g2 = _tc2(dpair, s1a[:N], s1b[:N], g1a, g1b, b1.reshape(1, D_HID), W2)
    s20, s21 = _sc_agg_out(g2, srcw, dstw, zer_out)
    z = _tc3(dpair, s20[:N], s21[:N], g2, b2.reshape(1, D_OUT))
    return z


# speedup vs baseline: 3.0680x; 1.3325x over previous
import functools

import jax
import jax.numpy as jnp
from jax import lax
from jax.experimental import pallas as pl
from jax.experimental.pallas import tpu as pltpu
from jax.experimental.pallas import tpu_sc as plsc

N, E, D_IN, D_HID, D_OUT = 10000, 320000, 128, 256, 128

CHUNK = 100
NSUB = 16
NCORE = 2
NCH = E // NSUB // CHUNK
GB = 10
NGRP = NCH // GB
NCHW = E // (NSUB * NCORE) // CHUNK

ACC_ROWS = 10112
RPS = ACC_ROWS // NSUB
DEG_ROWS = 10240
DPS = DEG_ROWS // NSUB

_mesh = functools.partial(
    plsc.VectorSubcoreMesh, core_axis_name="c", subcore_axis_name="s")


def _gs_pipe(g_hbm, acc, src_v, dst_v, bufs, gsem, ssem, nch):
    gcp = [None] * nch
    scp = [None] * nch
    gcp[0] = pltpu.async_copy(g_hbm.at[src_v.at[0]], bufs[0], gsem)
    if nch > 1:
        gcp[1] = pltpu.async_copy(g_hbm.at[src_v.at[1]], bufs[1], gsem)
    for j in range(nch):
        gcp[j].wait()
        scp[j] = pltpu.async_copy(bufs[j % 3], acc.at[dst_v.at[j]], ssem,
                                  add=True)
        if j + 2 < nch:
            if j >= 1:
                scp[j - 1].wait()
            gcp[j + 2] = pltpu.async_copy(
                g_hbm.at[src_v.at[j + 2]], bufs[(j + 2) % 3], gsem)
        elif j >= 1:
            scp[j - 1].wait()
    scp[nch - 1].wait()


@functools.partial(
    pl.kernel,
    out_type=jax.ShapeDtypeStruct((NCORE, DEG_ROWS), jnp.float32),
    mesh=_mesh(),
    scratch_types=[
        pltpu.VMEM_SHARED((DEG_ROWS,), jnp.float32),
        pltpu.VMEM((NCHW, CHUNK), jnp.int32),
        pltpu.VMEM((CHUNK,), jnp.float32),
    ],
)
def _sc_degree(dst_hbm, zer_hbm, ones_hbm, out_hbm, acc, idx_v, ones_v):
    cid = lax.axis_index("c")
    sid = lax.axis_index("s")
    wid = sid * NCORE + cid
    pltpu.sync_copy(zer_hbm, acc.at[pl.ds(sid * DPS, DPS)])
    pltpu.sync_copy(dst_hbm.at[wid], idx_v)
    pltpu.sync_copy(ones_hbm, ones_v)
    plsc.subcore_barrier()

    @pl.loop(0, NCHW)
    def _(i):
        pltpu.sync_copy(ones_v, acc.at[idx_v.at[i]], add=True)

    plsc.subcore_barrier()
    pltpu.sync_copy(acc.at[pl.ds(sid * DPS, DPS)],
                    out_hbm.at[cid, pl.ds(sid * DPS, DPS)])


def _make_sc_aggregate(dh):
    @functools.partial(
        pl.kernel,
        out_type=(jax.ShapeDtypeStruct((ACC_ROWS, dh), jnp.float32),
                  jax.ShapeDtypeStruct((ACC_ROWS, dh), jnp.float32)),
        mesh=_mesh(),
        scratch_types=[
            pltpu.VMEM_SHARED((ACC_ROWS, dh), jnp.float32),
            pltpu.VMEM((GB, CHUNK), jnp.int32),
            pltpu.VMEM((GB, CHUNK), jnp.int32),
            pltpu.VMEM((CHUNK, dh), jnp.float32),
            pltpu.VMEM((CHUNK, dh), jnp.float32),
            pltpu.VMEM((CHUNK, dh), jnp.float32),
            pltpu.SemaphoreType.DMA,
            pltpu.SemaphoreType.DMA,
        ],
    )
    def agg(ga_hbm, gb_hbm, src_hbm, dst_hbm, zer_hbm, outa_hbm, outb_hbm,
            acc, src_v, dst_v, rows0, rows1, rows2, gsem, ssem):
        cid = lax.axis_index("c")
        sid = lax.axis_index("s")
        pltpu.sync_copy(zer_hbm, acc.at[pl.ds(sid * RPS, RPS)])
        plsc.subcore_barrier()

        def half(g_hbm, out_hbm):
            @pl.loop(0, NGRP)
            def _(g):
                pltpu.sync_copy(src_hbm.at[sid, g], src_v)
                pltpu.sync_copy(dst_hbm.at[sid, g], dst_v)
                _gs_pipe(g_hbm, acc, src_v, dst_v, (rows0, rows1, rows2),
                         gsem, ssem, GB)

            plsc.subcore_barrier()
            pltpu.sync_copy(acc.at[pl.ds(sid * RPS, RPS)],
                            out_hbm.at[pl.ds(sid * RPS, RPS)])

        @pl.when(cid == 0)
        def _():
            half(ga_hbm, outa_hbm)

        @pl.when(cid == 1)
        def _():
            half(gb_hbm, outb_hbm)

    return agg


_sc_agg_hid = _make_sc_aggregate(D_HID // 2)

GB2 = 10
NGRP2 = NCHW // GB2


@functools.partial(
    pl.kernel,
    out_type=(jax.ShapeDtypeStruct((ACC_ROWS, D_OUT), jnp.float32),
              jax.ShapeDtypeStruct((ACC_ROWS, D_OUT), jnp.float32)),
    mesh=_mesh(),
    scratch_types=[
        pltpu.VMEM_SHARED((ACC_ROWS, D_OUT), jnp.float32),
        pltpu.VMEM((GB2, CHUNK), jnp.int32),
        pltpu.VMEM((GB2, CHUNK), jnp.int32),
        pltpu.VMEM((CHUNK, D_OUT), jnp.float32),
        pltpu.VMEM((CHUNK, D_OUT), jnp.float32),
        pltpu.VMEM((CHUNK, D_OUT), jnp.float32),
        pltpu.SemaphoreType.DMA,
        pltpu.SemaphoreType.DMA,
    ],
)
def _sc_agg_out(g_hbm, src_hbm, dst_hbm, zer_hbm, out0_hbm, out1_hbm,
                acc, src_v, dst_v, rows0, rows1, rows2, gsem, ssem):
    cid = lax.axis_index("c")
    sid = lax.axis_index("s")
    wid = sid * NCORE + cid
    pltpu.sync_copy(zer_hbm, acc.at[pl.ds(sid * RPS, RPS)])
    plsc.subcore_barrier()

    @pl.loop(0, NGRP2)
    def _(g):
        pltpu.sync_copy(src_hbm.at[wid, g], src_v)
        pltpu.sync_copy(dst_hbm.at[wid, g], dst_v)
        _gs_pipe(g_hbm, acc, src_v, dst_v, (rows0, rows1, rows2),
                 gsem, ssem, GB2)

    plsc.subcore_barrier()

    @pl.when(cid == 0)
    def _():
        pltpu.sync_copy(acc.at[pl.ds(sid * RPS, RPS)],
                        out0_hbm.at[pl.ds(sid * RPS, RPS)])

    @pl.when(cid == 1)
    def _():
        pltpu.sync_copy(acc.at[pl.ds(sid * RPS, RPS)],
                        out1_hbm.at[pl.ds(sid * RPS, RPS)])


_BR = 400
_GRID = N // _BR


def _dis_of(d_ref):
    return lax.rsqrt(d_ref[:, 0:1] + d_ref[:, 1:2] + 1.0)


def _tc1_body(d_ref, x_ref, w_ref, ga_ref, gb_ref):
    dis = _dis_of(d_ref)
    g = jnp.dot(x_ref[:], w_ref[:], preferred_element_type=jnp.float32) * dis
    ga_ref[:] = g[:, :D_HID // 2]
    gb_ref[:] = g[:, D_HID // 2:]


def _tc2_body(d_ref, sa_ref, sb_ref, ga_ref, gb_ref, b_ref, w_ref, o_ref):
    dis = _dis_of(d_ref)
    ha = jnp.maximum((sa_ref[:] + ga_ref[:]) * dis + b_ref[0, :D_HID // 2], 0.0)
    hb = jnp.maximum((sb_ref[:] + gb_ref[:]) * dis + b_ref[0, D_HID // 2:], 0.0)
    h = jnp.concatenate([ha, hb], axis=1)
    o_ref[:] = jnp.dot(h, w_ref[:], preferred_element_type=jnp.float32) * dis


def _tc3_body(d_ref, s0_ref, s1_ref, g_ref, b_ref, z_ref):
    dis = _dis_of(d_ref)
    z_ref[:] = (s0_ref[:] + s1_ref[:] + g_ref[:]) * dis + b_ref[0, :]


def _row_spec(c):
    return pl.BlockSpec((_BR, c), lambda i: (i, 0))


def _full_spec(r, c):
    return pl.BlockSpec((r, c), lambda i: (0, 0))


_tc1 = pl.pallas_call(
    _tc1_body,
    grid=(_GRID,),
    in_specs=[_row_spec(2), _row_spec(D_IN), _full_spec(D_IN, D_HID)],
    out_specs=[_row_spec(D_HID // 2)] * 2,
    out_shape=[jax.ShapeDtypeStruct((N, D_HID // 2), jnp.float32)] * 2,
)

_tc2 = pl.pallas_call(
    _tc2_body,
    grid=(_GRID,),
    in_specs=[_row_spec(2)] + [_row_spec(D_HID // 2)] * 4
    + [_full_spec(1, D_HID), _full_spec(D_HID, D_OUT)],
    out_specs=_row_spec(D_OUT),
    out_shape=jax.ShapeDtypeStruct((N, D_OUT), jnp.float32),
)

_tc3 = pl.pallas_call(
    _tc3_body,
    grid=(_GRID,),
    in_specs=[_row_spec(2)] + [_row_spec(D_OUT)] * 3
    + [_full_spec(1, D_OUT)],
    out_specs=_row_spec(D_OUT),
    out_shape=jax.ShapeDtypeStruct((N, D_OUT), jnp.float32),
)


def kernel(x, edge_index, W1, b1, W2, b2):
    src = edge_index[0].astype(jnp.int32)
    dst = edge_index[1].astype(jnp.int32)
    src3 = src.reshape(NSUB, NGRP, GB, CHUNK)
    dst3 = dst.reshape(NSUB, NGRP, GB, CHUNK)
    srcw = src.reshape(NSUB * NCORE, NGRP2, GB2, CHUNK)
    dstw = dst.reshape(NSUB * NCORE, NGRP2, GB2, CHUNK)
    dstd = dst.reshape(NSUB * NCORE, NCHW, CHUNK)

    zer_deg = jnp.zeros((DPS,), jnp.float32)
    ones_c = jnp.ones((CHUNK,), jnp.float32)
    zer_hid = jnp.zeros((RPS, D_HID // 2), jnp.float32)
    zer_out = jnp.zeros((RPS, D_OUT), jnp.float32)

    deg2 = _sc_degree(dstd, zer_deg, ones_c)
    dpair = deg2.T[:N]

    g1a, g1b = _tc1(dpair, x, W1)
    s1a, s1b = _sc_agg_hid(g1a, g1b, src3, dst3, zer_hid)
    g2 = _tc2(dpair, s1a[:N], s1b[:N], g1a, g1b, b1.reshape(1, D_HID), W2)
    s20, s21 = _sc_agg_out(g2, srcw, dstw, zer_out)
    z = _tc3(dpair, s20[:N], s21[:N], g2, b2.reshape(1, D_OUT))
    return z


# speedup vs baseline: 3.1728x; 1.0342x over previous
import functools

import jax
import jax.numpy as jnp
from jax import lax
from jax.experimental import pallas as pl
from jax.experimental.pallas import tpu as pltpu
from jax.experimental.pallas import tpu_sc as plsc

N, E, D_IN, D_HID, D_OUT = 10000, 320000, 128, 256, 128

CHUNK = 100
NSUB = 16
NCORE = 2
NCH = E // NSUB // CHUNK
GB = 10
NGRP = NCH // GB
NCHW = E // (NSUB * NCORE) // CHUNK

ACC_ROWS = 10112
RPS = ACC_ROWS // NSUB
DEG_ROWS = 10240
DPS = DEG_ROWS // NSUB

_mesh = functools.partial(
    plsc.VectorSubcoreMesh, core_axis_name="c", subcore_axis_name="s")


def _gs_pipe(g_hbm, acc, src_v, dst_v, bufs, gsem, ssem, nch):
    gcp = [None] * nch
    scp = [None] * nch
    gcp[0] = pltpu.async_copy(g_hbm.at[src_v.at[0]], bufs[0], gsem)
    if nch > 1:
        gcp[1] = pltpu.async_copy(g_hbm.at[src_v.at[1]], bufs[1], gsem)
    for j in range(nch):
        gcp[j].wait()
        scp[j] = pltpu.async_copy(bufs[j % 3], acc.at[dst_v.at[j]], ssem,
                                  add=True)
        if j + 2 < nch:
            if j >= 1:
                scp[j - 1].wait()
            gcp[j + 2] = pltpu.async_copy(
                g_hbm.at[src_v.at[j + 2]], bufs[(j + 2) % 3], gsem)
        elif j >= 1:
            scp[j - 1].wait()
    scp[nch - 1].wait()


@functools.partial(
    pl.kernel,
    out_type=jax.ShapeDtypeStruct((NCORE, DEG_ROWS), jnp.float32),
    mesh=_mesh(),
    scratch_types=[
        pltpu.VMEM_SHARED((DEG_ROWS,), jnp.float32),
        pltpu.VMEM((NCHW, CHUNK), jnp.int32),
        pltpu.VMEM((CHUNK,), jnp.float32),
    ],
)
def _sc_degree(dst_hbm, zer_hbm, ones_hbm, out_hbm, acc, idx_v, ones_v):
    cid = lax.axis_index("c")
    sid = lax.axis_index("s")
    wid = sid * NCORE + cid
    pltpu.sync_copy(zer_hbm, acc.at[pl.ds(sid * DPS, DPS)])
    pltpu.sync_copy(dst_hbm.at[wid], idx_v)
    pltpu.sync_copy(ones_hbm, ones_v)
    plsc.subcore_barrier()

    @pl.loop(0, NCHW)
    def _(i):
        pltpu.sync_copy(ones_v, acc.at[idx_v.at[i]], add=True)

    plsc.subcore_barrier()
    pltpu.sync_copy(acc.at[pl.ds(sid * DPS, DPS)],
                    out_hbm.at[cid, pl.ds(sid * DPS, DPS)])


def _make_sc_aggregate(dh):
    @functools.partial(
        pl.kernel,
        out_type=(jax.ShapeDtypeStruct((ACC_ROWS, dh), jnp.float32),
                  jax.ShapeDtypeStruct((ACC_ROWS, dh), jnp.float32)),
        mesh=_mesh(),
        scratch_types=[
            pltpu.VMEM_SHARED((ACC_ROWS, dh), jnp.float32),
            pltpu.VMEM((GB, CHUNK), jnp.int32),
            pltpu.VMEM((GB, CHUNK), jnp.int32),
            pltpu.VMEM((CHUNK, dh), jnp.float32),
            pltpu.VMEM((CHUNK, dh), jnp.float32),
            pltpu.VMEM((CHUNK, dh), jnp.float32),
            pltpu.SemaphoreType.DMA,
            pltpu.SemaphoreType.DMA,
        ],
    )
    def agg(ga_hbm, gb_hbm, src_hbm, dst_hbm, zer_hbm, outa_hbm, outb_hbm,
            acc, src_v, dst_v, rows0, rows1, rows2, gsem, ssem):
        cid = lax.axis_index("c")
        sid = lax.axis_index("s")
        pltpu.sync_copy(zer_hbm, acc.at[pl.ds(sid * RPS, RPS)])
        plsc.subcore_barrier()

        def half(g_hbm, out_hbm):
            @pl.loop(0, NGRP)
            def _(g):
                pltpu.sync_copy(src_hbm.at[sid, g], src_v)
                pltpu.sync_copy(dst_hbm.at[sid, g], dst_v)
                _gs_pipe(g_hbm, acc, src_v, dst_v, (rows0, rows1, rows2),
                         gsem, ssem, GB)

            plsc.subcore_barrier()
            pltpu.sync_copy(acc.at[pl.ds(sid * RPS, RPS)],
                            out_hbm.at[pl.ds(sid * RPS, RPS)])

        @pl.when(cid == 0)
        def _():
            half(ga_hbm, outa_hbm)

        @pl.when(cid == 1)
        def _():
            half(gb_hbm, outb_hbm)

    return agg


_sc_agg_hid = _make_sc_aggregate(D_HID // 2)

GB2 = 10
NGRP2 = NCHW // GB2


@functools.partial(
    pl.kernel,
    out_type=(jax.ShapeDtypeStruct((ACC_ROWS, D_OUT), jnp.float32),
              jax.ShapeDtypeStruct((ACC_ROWS, D_OUT), jnp.float32)),
    mesh=_mesh(),
    scratch_types=[
        pltpu.VMEM_SHARED((ACC_ROWS, D_OUT), jnp.float32),
        pltpu.VMEM((GB2, CHUNK), jnp.int32),
        pltpu.VMEM((GB2, CHUNK), jnp.int32),
        pltpu.VMEM((CHUNK, D_OUT), jnp.float32),
        pltpu.VMEM((CHUNK, D_OUT), jnp.float32),
        pltpu.VMEM((CHUNK, D_OUT), jnp.float32),
        pltpu.SemaphoreType.DMA,
        pltpu.SemaphoreType.DMA,
    ],
)
def _sc_agg_out(g_hbm, src_hbm, dst_hbm, zer_hbm, out0_hbm, out1_hbm,
                acc, src_v, dst_v, rows0, rows1, rows2, gsem, ssem):
    cid = lax.axis_index("c")
    sid = lax.axis_index("s")
    wid = sid * NCORE + cid
    pltpu.sync_copy(zer_hbm, acc.at[pl.ds(sid * RPS, RPS)])
    plsc.subcore_barrier()

    @pl.loop(0, NGRP2)
    def _(g):
        pltpu.sync_copy(src_hbm.at[wid, g], src_v)
        pltpu.sync_copy(dst_hbm.at[wid, g], dst_v)
        _gs_pipe(g_hbm, acc, src_v, dst_v, (rows0, rows1, rows2),
                 gsem, ssem, GB2)

    plsc.subcore_barrier()

    @pl.when(cid == 0)
    def _():
        pltpu.sync_copy(acc.at[pl.ds(sid * RPS, RPS)],
                        out0_hbm.at[pl.ds(sid * RPS, RPS)])

    @pl.when(cid == 1)
    def _():
        pltpu.sync_copy(acc.at[pl.ds(sid * RPS, RPS)],
                        out1_hbm.at[pl.ds(sid * RPS, RPS)])


_BR = 400
_GRID = N // _BR


def _dis_of(d_ref):
    return lax.rsqrt(d_ref[:, 0:1] + d_ref[:, 1:2] + 1.0)


def _tc1_body(d_ref, x_ref, w_ref, ga_ref, gb_ref):
    dis = _dis_of(d_ref)
    g = jnp.dot(x_ref[:], w_ref[:], preferred_element_type=jnp.float32) * dis
    ga_ref[:] = g[:, :D_HID // 2]
    gb_ref[:] = g[:, D_HID // 2:]


def _tc2_body(d_ref, sa_ref, sb_ref, ga_ref, gb_ref, b_ref, w_ref, o_ref):
    dis = _dis_of(d_ref)
    ha = jnp.maximum((sa_ref[:] + ga_ref[:]) * dis + b_ref[0, :D_HID // 2], 0.0)
    hb = jnp.maximum((sb_ref[:] + gb_ref[:]) * dis + b_ref[0, D_HID // 2:], 0.0)
    h = jnp.concatenate([ha, hb], axis=1)
    o_ref[:] = jnp.dot(h, w_ref[:], preferred_element_type=jnp.float32) * dis


def _tc3_body(d_ref, s0_ref, s1_ref, g_ref, b_ref, z_ref):
    dis = _dis_of(d_ref)
    z_ref[:] = (s0_ref[:] + s1_ref[:] + g_ref[:]) * dis + b_ref[0, :]


def _row_spec(c):
    return pl.BlockSpec((_BR, c), lambda i: (i, 0))


_DEG_SPEC = pl.BlockSpec((_BR, 2), lambda i: (i, 0))


def _full_spec(r, c):
    return pl.BlockSpec((r, c), lambda i: (0, 0))


_tc1 = pl.pallas_call(
    _tc1_body,
    grid=(_GRID,),
    in_specs=[_DEG_SPEC, _row_spec(D_IN), _full_spec(D_IN, D_HID)],
    out_specs=[_row_spec(D_HID // 2)] * 2,
    out_shape=[jax.ShapeDtypeStruct((N, D_HID // 2), jnp.float32)] * 2,
)

_tc2 = pl.pallas_call(
    _tc2_body,
    grid=(_GRID,),
    in_specs=[_DEG_SPEC] + [_row_spec(D_HID // 2)] * 4
    + [_full_spec(1, D_HID), _full_spec(D_HID, D_OUT)],
    out_specs=_row_spec(D_OUT),
    out_shape=jax.ShapeDtypeStruct((N, D_OUT), jnp.float32),
)

_tc3 = pl.pallas_call(
    _tc3_body,
    grid=(_GRID,),
    in_specs=[_DEG_SPEC] + [_row_spec(D_OUT)] * 3
    + [_full_spec(1, D_OUT)],
    out_specs=_row_spec(D_OUT),
    out_shape=jax.ShapeDtypeStruct((N, D_OUT), jnp.float32),
)


def kernel(x, edge_index, W1, b1, W2, b2):
    src = edge_index[0].astype(jnp.int32)
    dst = edge_index[1].astype(jnp.int32)
    src3 = src.reshape(NSUB, NGRP, GB, CHUNK)
    dst3 = dst.reshape(NSUB, NGRP, GB, CHUNK)
    srcw = src.reshape(NSUB * NCORE, NGRP2, GB2, CHUNK)
    dstw = dst.reshape(NSUB * NCORE, NGRP2, GB2, CHUNK)
    dstd = dst.reshape(NSUB * NCORE, NCHW, CHUNK)

    zer_deg = jnp.zeros((DPS,), jnp.float32)
    ones_c = jnp.ones((CHUNK,), jnp.float32)
    zer_hid = jnp.zeros((RPS, D_HID // 2), jnp.float32)
    zer_out = jnp.zeros((RPS, D_OUT), jnp.float32)

    deg2 = _sc_degree(dstd, zer_deg, ones_c)
    dpair = deg2.T[:N]

    g1a, g1b = _tc1(dpair, x, W1)
    s1a, s1b = _sc_agg_hid(g1a, g1b, src3, dst3, zer_hid)
    g2 = _tc2(dpair, s1a, s1b, g1a, g1b, b1.reshape(1, D_HID), W2)
    s20, s21 = _sc_agg_out(g2, srcw, dstw, zer_out)
    z = _tc3(dpair, s20, s21, g2, b2.reshape(1, D_OUT))
    return z


# speedup vs baseline: 3.4000x; 1.0716x over previous
import functools

import jax
import jax.numpy as jnp
from jax import lax
from jax.experimental import pallas as pl
from jax.experimental.pallas import tpu as pltpu
from jax.experimental.pallas import tpu_sc as plsc

N, E, D_IN, D_HID, D_OUT = 10000, 320000, 128, 256, 128

CHUNK = 100
NSUB = 16
NCORE = 2
NCH = E // NSUB // CHUNK
GB = 20
NGRP = NCH // GB
NCHW = E // (NSUB * NCORE) // CHUNK

ACC_ROWS = 10112
RPS = ACC_ROWS // NSUB
DEG_ROWS = 10240
DPS = DEG_ROWS // NSUB

_mesh = functools.partial(
    plsc.VectorSubcoreMesh, core_axis_name="c", subcore_axis_name="s")


def _gs_pipe(g_hbm, acc, src_v, dst_v, bufs, gsem, ssem, nch):
    gcp = [None] * nch
    scp = [None] * nch
    gcp[0] = pltpu.async_copy(g_hbm.at[src_v.at[0]], bufs[0], gsem)
    if nch > 1:
        gcp[1] = pltpu.async_copy(g_hbm.at[src_v.at[1]], bufs[1], gsem)
    for j in range(nch):
        gcp[j].wait()
        scp[j] = pltpu.async_copy(bufs[j % 3], acc.at[dst_v.at[j]], ssem,
                                  add=True)
        if j + 2 < nch:
            if j >= 1:
                scp[j - 1].wait()
            gcp[j + 2] = pltpu.async_copy(
                g_hbm.at[src_v.at[j + 2]], bufs[(j + 2) % 3], gsem)
        elif j >= 1:
            scp[j - 1].wait()
    scp[nch - 1].wait()


@functools.partial(
    pl.kernel,
    out_type=jax.ShapeDtypeStruct((NCORE, DEG_ROWS), jnp.float32),
    mesh=_mesh(),
    scratch_types=[
        pltpu.VMEM_SHARED((DEG_ROWS,), jnp.float32),
        pltpu.VMEM((NCHW, CHUNK), jnp.int32),
        pltpu.VMEM((CHUNK,), jnp.float32),
    ],
)
def _sc_degree(dst_hbm, zer_hbm, ones_hbm, out_hbm, acc, idx_v, ones_v):
    cid = lax.axis_index("c")
    sid = lax.axis_index("s")
    wid = sid * NCORE + cid
    pltpu.sync_copy(zer_hbm, acc.at[pl.ds(sid * DPS, DPS)])
    pltpu.sync_copy(dst_hbm.at[wid], idx_v)
    pltpu.sync_copy(ones_hbm, ones_v)
    plsc.subcore_barrier()

    @pl.loop(0, NCHW)
    def _(i):
        pltpu.sync_copy(ones_v, acc.at[idx_v.at[i]], add=True)

    plsc.subcore_barrier()
    pltpu.sync_copy(acc.at[pl.ds(sid * DPS, DPS)],
                    out_hbm.at[cid, pl.ds(sid * DPS, DPS)])


def _make_sc_aggregate(dh):
    @functools.partial(
        pl.kernel,
        out_type=(jax.ShapeDtypeStruct((ACC_ROWS, dh), jnp.float32),
                  jax.ShapeDtypeStruct((ACC_ROWS, dh), jnp.float32)),
        mesh=_mesh(),
        scratch_types=[
            pltpu.VMEM_SHARED((ACC_ROWS, dh), jnp.float32),
            pltpu.VMEM((GB, CHUNK), jnp.int32),
            pltpu.VMEM((GB, CHUNK), jnp.int32),
            pltpu.VMEM((CHUNK, dh), jnp.float32),
            pltpu.VMEM((CHUNK, dh), jnp.float32),
            pltpu.VMEM((CHUNK, dh), jnp.float32),
            pltpu.SemaphoreType.DMA,
            pltpu.SemaphoreType.DMA,
        ],
    )
    def agg(ga_hbm, gb_hbm, src_hbm, dst_hbm, zer_hbm, outa_hbm, outb_hbm,
            acc, src_v, dst_v, rows0, rows1, rows2, gsem, ssem):
        cid = lax.axis_index("c")
        sid = lax.axis_index("s")
        pltpu.sync_copy(zer_hbm, acc.at[pl.ds(sid * RPS, RPS)])
        plsc.subcore_barrier()

        def half(g_hbm, out_hbm):
            @pl.loop(0, NGRP)
            def _(g):
                pltpu.sync_copy(src_hbm.at[sid, g], src_v)
                pltpu.sync_copy(dst_hbm.at[sid, g], dst_v)
                _gs_pipe(g_hbm, acc, src_v, dst_v, (rows0, rows1, rows2),
                         gsem, ssem, GB)

            plsc.subcore_barrier()
            pltpu.sync_copy(acc.at[pl.ds(sid * RPS, RPS)],
                            out_hbm.at[pl.ds(sid * RPS, RPS)])

        @pl.when(cid == 0)
        def _():
            half(ga_hbm, outa_hbm)

        @pl.when(cid == 1)
        def _():
            half(gb_hbm, outb_hbm)

    return agg


_sc_agg_hid = _make_sc_aggregate(D_HID // 2)

GB2 = 20
NGRP2 = NCHW // GB2


@functools.partial(
    pl.kernel,
    out_type=(jax.ShapeDtypeStruct((ACC_ROWS, D_OUT), jnp.float32),
              jax.ShapeDtypeStruct((ACC_ROWS, D_OUT), jnp.float32)),
    mesh=_mesh(),
    scratch_types=[
        pltpu.VMEM_SHARED((ACC_ROWS, D_OUT), jnp.float32),
        pltpu.VMEM((GB2, CHUNK), jnp.int32),
        pltpu.VMEM((GB2, CHUNK), jnp.int32),
        pltpu.VMEM((CHUNK, D_OUT), jnp.float32),
        pltpu.VMEM((CHUNK, D_OUT), jnp.float32),
        pltpu.VMEM((CHUNK, D_OUT), jnp.float32),
        pltpu.SemaphoreType.DMA,
        pltpu.SemaphoreType.DMA,
    ],
)
def _sc_agg_out(g_hbm, src_hbm, dst_hbm, zer_hbm, out0_hbm, out1_hbm,
                acc, src_v, dst_v, rows0, rows1, rows2, gsem, ssem):
    cid = lax.axis_index("c")
    sid = lax.axis_index("s")
    wid = sid * NCORE + cid
    pltpu.sync_copy(zer_hbm, acc.at[pl.ds(sid * RPS, RPS)])
    plsc.subcore_barrier()

    @pl.loop(0, NGRP2)
    def _(g):
        pltpu.sync_copy(src_hbm.at[wid, g], src_v)
        pltpu.sync_copy(dst_hbm.at[wid, g], dst_v)
        _gs_pipe(g_hbm, acc, src_v, dst_v, (rows0, rows1, rows2),
                 gsem, ssem, GB2)

    plsc.subcore_barrier()

    @pl.when(cid == 0)
    def _():
        pltpu.sync_copy(acc.at[pl.ds(sid * RPS, RPS)],
                        out0_hbm.at[pl.ds(sid * RPS, RPS)])

    @pl.when(cid == 1)
    def _():
        pltpu.sync_copy(acc.at[pl.ds(sid * RPS, RPS)],
                        out1_hbm.at[pl.ds(sid * RPS, RPS)])


_BR = 400
_GRID = N // _BR


def _dis_of(d_ref):
    return lax.rsqrt(d_ref[:, 0:1] + d_ref[:, 1:2] + 1.0)


def _tc1_body(d_ref, x_ref, w_ref, ga_ref, gb_ref):
    dis = _dis_of(d_ref)
    g = jnp.dot(x_ref[:], w_ref[:], preferred_element_type=jnp.float32) * dis
    ga_ref[:] = g[:, :D_HID // 2]
    gb_ref[:] = g[:, D_HID // 2:]


def _tc2_body(d_ref, sa_ref, sb_ref, ga_ref, gb_ref, b_ref, w_ref, o_ref):
    dis = _dis_of(d_ref)
    ha = jnp.maximum((sa_ref[:] + ga_ref[:]) * dis + b_ref[0, :D_HID // 2], 0.0)
    hb = jnp.maximum((sb_ref[:] + gb_ref[:]) * dis + b_ref[0, D_HID // 2:], 0.0)
    h = jnp.concatenate([ha, hb], axis=1)
    o_ref[:] = jnp.dot(h, w_ref[:], preferred_element_type=jnp.float32) * dis


def _tc3_body(d_ref, s0_ref, s1_ref, g_ref, b_ref, z_ref):
    dis = _dis_of(d_ref)
    z_ref[:] = (s0_ref[:] + s1_ref[:] + g_ref[:]) * dis + b_ref[0, :]


def _row_spec(c):
    return pl.BlockSpec((_BR, c), lambda i: (i, 0))


_DEG_SPEC = pl.BlockSpec((_BR, 2), lambda i: (i, 0))


def _full_spec(r, c):
    return pl.BlockSpec((r, c), lambda i: (0, 0))


_tc1 = pl.pallas_call(
    _tc1_body,
    grid=(_GRID,),
    in_specs=[_DEG_SPEC, _row_spec(D_IN), _full_spec(D_IN, D_HID)],
    out_specs=[_row_spec(D_HID // 2)] * 2,
    out_shape=[jax.ShapeDtypeStruct((N, D_HID // 2), jnp.float32)] * 2,
)

_tc2 = pl.pallas_call(
    _tc2_body,
    grid=(_GRID,),
    in_specs=[_DEG_SPEC] + [_row_spec(D_HID // 2)] * 4
    + [_full_spec(1, D_HID), _full_spec(D_HID, D_OUT)],
    out_specs=_row_spec(D_OUT),
    out_shape=jax.ShapeDtypeStruct((N, D_OUT), jnp.float32),
)

_tc3 = pl.pallas_call(
    _tc3_body,
    grid=(_GRID,),
    in_specs=[_DEG_SPEC] + [_row_spec(D_OUT)] * 3
    + [_full_spec(1, D_OUT)],
    out_specs=_row_spec(D_OUT),
    out_shape=jax.ShapeDtypeStruct((N, D_OUT), jnp.float32),
)


def kernel(x, edge_index, W1, b1, W2, b2):
    src = edge_index[0].astype(jnp.int32)
    dst = edge_index[1].astype(jnp.int32)
    src3 = src.reshape(NSUB, NGRP, GB, CHUNK)
    dst3 = dst.reshape(NSUB, NGRP, GB, CHUNK)
    srcw = src.reshape(NSUB * NCORE, NGRP2, GB2, CHUNK)
    dstw = dst.reshape(NSUB * NCORE, NGRP2, GB2, CHUNK)
    dstd = dst.reshape(NSUB * NCORE, NCHW, CHUNK)

    zer_deg = jnp.zeros((DPS,), jnp.float32)
    ones_c = jnp.ones((CHUNK,), jnp.float32)
    zer_hid = jnp.zeros((RPS, D_HID // 2), jnp.float32)
    zer_out = jnp.zeros((RPS, D_OUT), jnp.float32)

    deg2 = _sc_degree(dstd, zer_deg, ones_c)
    dpair = deg2.T[:N]

    g1a, g1b = _tc1(dpair, x, W1)
    s1a, s1b = _sc_agg_hid(g1a, g1b, src3, dst3, zer_hid)
    g2 = _tc2(dpair, s1a, s1b, g1a, g1b, b1.reshape(1, D_HID), W2)
    s20, s21 = _sc_agg_out(g2, srcw, dstw, zer_out)
    z = _tc3(dpair, s20, s21, g2, b2.reshape(1, D_OUT))
    return z


# speedup vs baseline: 3.4204x; 1.0060x over previous
import functools

import jax
import jax.numpy as jnp
from jax import lax
from jax.experimental import pallas as pl
from jax.experimental.pallas import tpu as pltpu
from jax.experimental.pallas import tpu_sc as plsc

N, E, D_IN, D_HID, D_OUT = 10000, 320000, 128, 256, 128

CHUNK = 100
NSUB = 16
NCORE = 2
NCH = E // NSUB // CHUNK
GB = 25
NGRP = NCH // GB
NCHW = E // (NSUB * NCORE) // CHUNK

ACC_ROWS = 10112
RPS = ACC_ROWS // NSUB
DEG_ROWS = 10240
DPS = DEG_ROWS // NSUB

_mesh = functools.partial(
    plsc.VectorSubcoreMesh, core_axis_name="c", subcore_axis_name="s")


def _gs_pipe(g_hbm, acc, src_v, dst_v, bufs, gsem, ssem, nch):
    gcp = [None] * nch
    scp = [None] * nch
    gcp[0] = pltpu.async_copy(g_hbm.at[src_v.at[0]], bufs[0], gsem)
    if nch > 1:
        gcp[1] = pltpu.async_copy(g_hbm.at[src_v.at[1]], bufs[1], gsem)
    for j in range(nch):
        gcp[j].wait()
        scp[j] = pltpu.async_copy(bufs[j % 3], acc.at[dst_v.at[j]], ssem,
                                  add=True)
        if j + 2 < nch:
            if j >= 1:
                scp[j - 1].wait()
            gcp[j + 2] = pltpu.async_copy(
                g_hbm.at[src_v.at[j + 2]], bufs[(j + 2) % 3], gsem)
        elif j >= 1:
            scp[j - 1].wait()
    scp[nch - 1].wait()


@functools.partial(
    pl.kernel,
    out_type=jax.ShapeDtypeStruct((NCORE, DEG_ROWS), jnp.float32),
    mesh=_mesh(),
    scratch_types=[
        pltpu.VMEM_SHARED((DEG_ROWS,), jnp.float32),
        pltpu.VMEM((NCHW, CHUNK), jnp.int32),
        pltpu.VMEM((CHUNK,), jnp.float32),
    ],
)
def _sc_degree(dst_hbm, zer_hbm, ones_hbm, out_hbm, acc, idx_v, ones_v):
    cid = lax.axis_index("c")
    sid = lax.axis_index("s")
    wid = sid * NCORE + cid
    pltpu.sync_copy(zer_hbm, acc.at[pl.ds(sid * DPS, DPS)])
    pltpu.sync_copy(dst_hbm.at[wid], idx_v)
    pltpu.sync_copy(ones_hbm, ones_v)
    plsc.subcore_barrier()

    @pl.loop(0, NCHW)
    def _(i):
        pltpu.sync_copy(ones_v, acc.at[idx_v.at[i]], add=True)

    plsc.subcore_barrier()
    pltpu.sync_copy(acc.at[pl.ds(sid * DPS, DPS)],
                    out_hbm.at[cid, pl.ds(sid * DPS, DPS)])


def _make_sc_aggregate(dh):
    @functools.partial(
        pl.kernel,
        out_type=(jax.ShapeDtypeStruct((ACC_ROWS, dh), jnp.float32),
                  jax.ShapeDtypeStruct((ACC_ROWS, dh), jnp.float32)),
        mesh=_mesh(),
        scratch_types=[
            pltpu.VMEM_SHARED((ACC_ROWS, dh), jnp.float32),
            pltpu.VMEM((GB, CHUNK), jnp.int32),
            pltpu.VMEM((GB, CHUNK), jnp.int32),
            pltpu.VMEM((CHUNK, dh), jnp.float32),
            pltpu.VMEM((CHUNK, dh), jnp.float32),
            pltpu.VMEM((CHUNK, dh), jnp.float32),
            pltpu.SemaphoreType.DMA,
            pltpu.SemaphoreType.DMA,
        ],
    )
    def agg(ga_hbm, gb_hbm, src_hbm, dst_hbm, zer_hbm, outa_hbm, outb_hbm,
            acc, src_v, dst_v, rows0, rows1, rows2, gsem, ssem):
        cid = lax.axis_index("c")
        sid = lax.axis_index("s")
        pltpu.sync_copy(zer_hbm, acc.at[pl.ds(sid * RPS, RPS)])
        plsc.subcore_barrier()

        def half(g_hbm, out_hbm):
            @pl.loop(0, NGRP)
            def _(g):
                pltpu.sync_copy(src_hbm.at[sid, g], src_v)
                pltpu.sync_copy(dst_hbm.at[sid, g], dst_v)
                _gs_pipe(g_hbm, acc, src_v, dst_v, (rows0, rows1, rows2),
                         gsem, ssem, GB)

            plsc.subcore_barrier()
            pltpu.sync_copy(acc.at[pl.ds(sid * RPS, RPS)],
                            out_hbm.at[pl.ds(sid * RPS, RPS)])

        @pl.when(cid == 0)
        def _():
            half(ga_hbm, outa_hbm)

        @pl.when(cid == 1)
        def _():
            half(gb_hbm, outb_hbm)

    return agg


_sc_agg_hid = _make_sc_aggregate(D_HID // 2)

GB2 = 25
NGRP2 = NCHW // GB2


@functools.partial(
    pl.kernel,
    out_type=(jax.ShapeDtypeStruct((ACC_ROWS, D_OUT), jnp.float32),
              jax.ShapeDtypeStruct((ACC_ROWS, D_OUT), jnp.float32)),
    mesh=_mesh(),
    scratch_types=[
        pltpu.VMEM_SHARED((ACC_ROWS, D_OUT), jnp.float32),
        pltpu.VMEM((GB2, CHUNK), jnp.int32),
        pltpu.VMEM((GB2, CHUNK), jnp.int32),
        pltpu.VMEM((CHUNK, D_OUT), jnp.float32),
        pltpu.VMEM((CHUNK, D_OUT), jnp.float32),
        pltpu.VMEM((CHUNK, D_OUT), jnp.float32),
        pltpu.SemaphoreType.DMA,
        pltpu.SemaphoreType.DMA,
    ],
)
def _sc_agg_out(g_hbm, src_hbm, dst_hbm, zer_hbm, out0_hbm, out1_hbm,
                acc, src_v, dst_v, rows0, rows1, rows2, gsem, ssem):
    cid = lax.axis_index("c")
    sid = lax.axis_index("s")
    wid = sid * NCORE + cid
    pltpu.sync_copy(zer_hbm, acc.at[pl.ds(sid * RPS, RPS)])
    plsc.subcore_barrier()

    @pl.loop(0, NGRP2)
    def _(g):
        pltpu.sync_copy(src_hbm.at[wid, g], src_v)
        pltpu.sync_copy(dst_hbm.at[wid, g], dst_v)
        _gs_pipe(g_hbm, acc, src_v, dst_v, (rows0, rows1, rows2),
                 gsem, ssem, GB2)

    plsc.subcore_barrier()

    @pl.when(cid == 0)
    def _():
        pltpu.sync_copy(acc.at[pl.ds(sid * RPS, RPS)],
                        out0_hbm.at[pl.ds(sid * RPS, RPS)])

    @pl.when(cid == 1)
    def _():
        pltpu.sync_copy(acc.at[pl.ds(sid * RPS, RPS)],
                        out1_hbm.at[pl.ds(sid * RPS, RPS)])


_BR = 400
_GRID = N // _BR


def _dis_of(d_ref):
    return lax.rsqrt(d_ref[:, 0:1] + d_ref[:, 1:2] + 1.0)


def _tc1_body(d_ref, x_ref, w_ref, ga_ref, gb_ref):
    dis = _dis_of(d_ref)
    g = jnp.dot(x_ref[:], w_ref[:], preferred_element_type=jnp.float32) * dis
    ga_ref[:] = g[:, :D_HID // 2]
    gb_ref[:] = g[:, D_HID // 2:]


def _tc2_body(d_ref, sa_ref, sb_ref, ga_ref, gb_ref, b_ref, w_ref, o_ref):
    dis = _dis_of(d_ref)
    ha = jnp.maximum((sa_ref[:] + ga_ref[:]) * dis + b_ref[0, :D_HID // 2], 0.0)
    hb = jnp.maximum((sb_ref[:] + gb_ref[:]) * dis + b_ref[0, D_HID // 2:], 0.0)
    h = jnp.concatenate([ha, hb], axis=1)
    o_ref[:] = jnp.dot(h, w_ref[:], preferred_element_type=jnp.float32) * dis


def _tc3_body(d_ref, s0_ref, s1_ref, g_ref, b_ref, z_ref):
    dis = _dis_of(d_ref)
    z_ref[:] = (s0_ref[:] + s1_ref[:] + g_ref[:]) * dis + b_ref[0, :]


def _row_spec(c):
    return pl.BlockSpec((_BR, c), lambda i: (i, 0))


_DEG_SPEC = pl.BlockSpec((_BR, 2), lambda i: (i, 0))


def _full_spec(r, c):
    return pl.BlockSpec((r, c), lambda i: (0, 0))


_tc1 = pl.pallas_call(
    _tc1_body,
    grid=(_GRID,),
    in_specs=[_DEG_SPEC, _row_spec(D_IN), _full_spec(D_IN, D_HID)],
    out_specs=[_row_spec(D_HID // 2)] * 2,
    out_shape=[jax.ShapeDtypeStruct((N, D_HID // 2), jnp.float32)] * 2,
)

_tc2 = pl.pallas_call(
    _tc2_body,
    grid=(_GRID,),
    in_specs=[_DEG_SPEC] + [_row_spec(D_HID // 2)] * 4
    + [_full_spec(1, D_HID), _full_spec(D_HID, D_OUT)],
    out_specs=_row_spec(D_OUT),
    out_shape=jax.ShapeDtypeStruct((N, D_OUT), jnp.float32),
)

_tc3 = pl.pallas_call(
    _tc3_body,
    grid=(_GRID,),
    in_specs=[_DEG_SPEC] + [_row_spec(D_OUT)] * 3
    + [_full_spec(1, D_OUT)],
    out_specs=_row_spec(D_OUT),
    out_shape=jax.ShapeDtypeStruct((N, D_OUT), jnp.float32),
)


def kernel(x, edge_index, W1, b1, W2, b2):
    src = edge_index[0].astype(jnp.int32)
    dst = edge_index[1].astype(jnp.int32)
    src3 = src.reshape(NSUB, NGRP, GB, CHUNK)
    dst3 = dst.reshape(NSUB, NGRP, GB, CHUNK)
    srcw = src.reshape(NSUB * NCORE, NGRP2, GB2, CHUNK)
    dstw = dst.reshape(NSUB * NCORE, NGRP2, GB2, CHUNK)
    dstd = dst.reshape(NSUB * NCORE, NCHW, CHUNK)

    zer_deg = jnp.zeros((DPS,), jnp.float32)
    ones_c = jnp.ones((CHUNK,), jnp.float32)
    zer_hid = jnp.zeros((RPS, D_HID // 2), jnp.float32)
    zer_out = jnp.zeros((RPS, D_OUT), jnp.float32)

    deg2 = _sc_degree(dstd, zer_deg, ones_c)
    dpair = deg2.T[:N]

    g1a, g1b = _tc1(dpair, x, W1)
    s1a, s1b = _sc_agg_hid(g1a, g1b, src3, dst3, zer_hid)
    g2 = _tc2(dpair, s1a, s1b, g1a, g1b, b1.reshape(1, D_HID), W2)
    s20, s21 = _sc_agg_out(g2, srcw, dstw, zer_out)
    z = _tc3(dpair, s20, s21, g2, b2.reshape(1, D_OUT))
    return z
